# baseline (speedup 1.0000x reference)
import functools

import jax
import jax.numpy as jnp
from jax import lax
from jax.experimental import pallas as pl
from jax.experimental.pallas import tpu as pltpu
from jax.experimental.pallas import tpu_sc as plsc

N = 10000
E = 160000
D = 256
G = 32
DH = 128
NC = 2
NS = 16
CH = 128
ROWS = E // CH
NBUF = 3
RQ = -(-(-(-ROWS // NS)) // NBUF) * NBUF
ROWS_PAD = RQ * NS
NPAD = 16
ZB = (N // NS) // 8 * 8
ZREM = N - ZB * NS
BN = 1000
GRID = N // BN


def _proj_body(h_ref, w_ref, b_ref, xl2_ref, gam_ref, bet_ref):
    h = h_ref[...]
    o = jnp.dot(h, w_ref[...], preferred_element_type=jnp.float32) + b_ref[...]
    xl2_ref[0] = o[:, :DH]
    xl2_ref[1] = o[:, DH:D]
    gam_ref[...] = o[:, D:D + D]
    bet_ref[...] = o[:, D + D:]


def _film_proj_body(gamp_ref, betp_ref, agg_ref, w_ref, b_ref,
                    xl2_ref, gam_ref, bet_ref):
    agg = jnp.concatenate([agg_ref[0], agg_ref[1]], axis=-1)
    h = jnp.maximum(gamp_ref[...] * agg + betp_ref[...], 0.0)
    o = jnp.dot(h, w_ref[...], preferred_element_type=jnp.float32) + b_ref[...]
    xl2_ref[0] = o[:, :DH]
    xl2_ref[1] = o[:, DH:D]
    gam_ref[...] = o[:, D:D + D]
    bet_ref[...] = o[:, D + D:]


def _film_pool_body(gamp_ref, betp_ref, agg_ref, batch_ref, out_ref,
                    sums_scr, counts_scr):
    i = pl.program_id(0)
    agg = jnp.concatenate([agg_ref[0], agg_ref[1]], axis=-1)
    h = jnp.maximum(gamp_ref[...] * agg + betp_ref[...], 0.0)
    bvec = batch_ref[0]
    gids = lax.broadcasted_iota(jnp.int32, (G, BN), 0)
    onehot = (gids == bvec).astype(jnp.float32)
    ps = jnp.dot(onehot, h, preferred_element_type=jnp.float32)
    pc = jnp.dot(onehot, jnp.ones((BN, D), jnp.float32),
                 preferred_element_type=jnp.float32)

    @pl.when(i == 0)
    def _():
        sums_scr[...] = ps
        counts_scr[...] = pc

    @pl.when(i > 0)
    def _():
        sums_scr[...] += ps
        counts_scr[...] += pc

    @pl.when(i == pl.num_programs(0) - 1)
    def _():
        out_ref[...] = sums_scr[...] / jnp.maximum(counts_scr[...], 1.0)


_W_SPEC = pl.BlockSpec((D, 3 * D), lambda i: (0, 0))
_B_SPEC = pl.BlockSpec((1, 3 * D), lambda i: (0, 0))
_H_SPEC = pl.BlockSpec((BN, D), lambda i: (i, 0))
_XL2_SPEC = pl.BlockSpec((2, BN, DH), lambda i: (0, i, 0))

_proj_call = pl.pallas_call(
    _proj_body,
    grid=(GRID,),
    in_specs=[_H_SPEC, _W_SPEC, _B_SPEC],
    out_specs=[_XL2_SPEC, _H_SPEC, _H_SPEC],
    out_shape=[
        jax.ShapeDtypeStruct((2, N, DH), jnp.float32),
        jax.ShapeDtypeStruct((N, D), jnp.float32),
        jax.ShapeDtypeStruct((N, D), jnp.float32),
    ],
)

_film_proj_call = pl.pallas_call(
    _film_proj_body,
    grid=(GRID,),
    in_specs=[_H_SPEC, _H_SPEC, _XL2_SPEC, _W_SPEC, _B_SPEC],
    out_specs=[_XL2_SPEC, _H_SPEC, _H_SPEC],
    out_shape=[
        jax.ShapeDtypeStruct((2, N, DH), jnp.float32),
        jax.ShapeDtypeStruct((N, D), jnp.float32),
        jax.ShapeDtypeStruct((N, D), jnp.float32),
    ],
)

_film_pool_call = pl.pallas_call(
    _film_pool_body,
    grid=(GRID,),
    in_specs=[_H_SPEC, _H_SPEC, _XL2_SPEC,
              pl.BlockSpec((1, 1, BN), lambda i: (i, 0, 0))],
    out_specs=pl.BlockSpec((G, D), lambda i: (0, 0)),
    out_shape=jax.ShapeDtypeStruct((G, D), jnp.float32),
    scratch_shapes=[pltpu.VMEM((G, D), jnp.float32),
                    pltpu.VMEM((G, D), jnp.float32)],
)


def _edge_body(xl_hbm, srcr_hbm, dstr_hbm, zeros_hbm, out_hbm,
               sidx, didx, rows, acc, isem, dsem, gsem, ssem):
    c = lax.axis_index("c")
    s = lax.axis_index("s")

    pltpu.sync_copy(zeros_hbm, acc.at[pl.ds(s * ZB, ZB)])

    @pl.when(s == NS - 1)
    def _():
        pltpu.sync_copy(zeros_hbm.at[pl.ds(0, ZREM + NPAD)],
                        acc.at[pl.ds(N - ZREM, ZREM + NPAD)])

    rowbase = s * RQ
    plsc.subcore_barrier()

    def launch(k, slot, drain):
        if drain:
            pltpu.make_async_copy(zeros_hbm.at[pl.ds(0, CH)],
                                  rows.at[slot], ssem[slot]).wait()
        pltpu.make_async_copy(srcr_hbm.at[pl.ds(rowbase + k, 1)],
                              sidx[slot], isem[slot]).wait()
        pltpu.async_copy(xl_hbm.at[c].at[sidx[slot].at[0]],
                         rows.at[slot], gsem[slot])
        pltpu.async_copy(dstr_hbm.at[pl.ds(rowbase + k, 1)],
                         didx[slot], dsem[slot])

    def consume(j, slot, prefetch):
        pltpu.make_async_copy(xl_hbm.at[c].at[sidx[slot].at[0]],
                              rows.at[slot], gsem[slot]).wait()
        pltpu.make_async_copy(dstr_hbm.at[pl.ds(rowbase + j, 1)],
                              didx[slot], dsem[slot]).wait()
        pltpu.async_copy(rows.at[slot], acc.at[didx[slot].at[0]],
                         ssem[slot], add=True)
        if prefetch:
            pltpu.async_copy(srcr_hbm.at[pl.ds(rowbase + j + NBUF, 1)],
                             sidx[slot], isem[slot])

    for b in range(NBUF):
        pltpu.async_copy(srcr_hbm.at[pl.ds(rowbase + b, 1)],
                         sidx[b], isem[b])
    launch(0, 0, drain=False)

    for b in range(NBUF):
        launch(b + 1, (b + 1) % NBUF, drain=(b + 1 >= NBUF))
        consume(b, b, prefetch=True)

    def body(g, carry):
        for b in range(NBUF):
            j = g * NBUF + b
            jn = j + 1

            @pl.when(jn < RQ)
            def _():
                launch(jn, (b + 1) % NBUF, drain=True)

            @pl.when(j + NBUF < RQ)
            def _():
                consume(j, b, prefetch=True)

            @pl.when(j + NBUF >= RQ)
            def _():
                consume(j, b, prefetch=False)
        return carry

    lax.fori_loop(1, RQ // NBUF, body, 0)

    for b in range(NBUF):
        pltpu.make_async_copy(zeros_hbm.at[pl.ds(0, CH)],
                              rows.at[b], ssem[b]).wait()

    plsc.subcore_barrier()

    pltpu.sync_copy(acc.at[pl.ds(s * ZB, ZB)],
                    out_hbm.at[c, pl.ds(s * ZB, ZB)])

    @pl.when(s == NS - 1)
    def _():
        pltpu.sync_copy(acc.at[pl.ds(N - ZREM, ZREM)],
                        out_hbm.at[c, pl.ds(N - ZREM, ZREM)])


@functools.cache
def _get_edge_call():
    return pl.kernel(
        _edge_body,
        out_type=jax.ShapeDtypeStruct((2, N, DH), jnp.float32),
        mesh=plsc.VectorSubcoreMesh(core_axis_name="c", subcore_axis_name="s",
                                    num_cores=NC, num_subcores=NS),
        scratch_types=[
            [pltpu.VMEM((1, CH), jnp.int32)] * NBUF,
            [pltpu.VMEM((1, CH), jnp.int32)] * NBUF,
            pltpu.VMEM((NBUF, CH, DH), jnp.float32),
            pltpu.VMEM_SHARED((N + NPAD, DH), jnp.float32),
            [pltpu.SemaphoreType.DMA] * NBUF,
            [pltpu.SemaphoreType.DMA] * NBUF,
            [pltpu.SemaphoreType.DMA] * NBUF,
            [pltpu.SemaphoreType.DMA] * NBUF,
        ],
    )


def _edge_call(xl2, src_r, dst_r, zeros):
    return _get_edge_call()(xl2, src_r, dst_r, zeros)


def kernel(x, edge_index, batch,
           W_lin0, b_lin0, W_gam0, b_gam0, W_bet0, b_bet0,
           W_lin1, b_lin1, W_gam1, b_gam1, W_bet1, b_bet1,
           W_lin2, b_lin2, W_gam2, b_gam2, W_bet2, b_bet2):
    npad_e = ROWS_PAD * CH - E
    src = edge_index[0].astype(jnp.int32)
    dst = edge_index[1].astype(jnp.int32)
    pad_src = jnp.zeros((npad_e,), jnp.int32)
    pad_dst = N + jnp.arange(npad_e, dtype=jnp.int32) % NPAD
    src_r = jnp.concatenate([src, pad_src]).reshape(ROWS_PAD, CH)
    dst_r = jnp.concatenate([dst, pad_dst]).reshape(ROWS_PAD, CH)
    zeros = jnp.zeros((ZB, DH), jnp.float32)
    batch3 = batch.astype(jnp.int32).reshape(GRID, 1, BN)

    params = []
    for (Wl, bl, Wg, bg, Wb, bb) in (
            (W_lin0, b_lin0, W_gam0, b_gam0, W_bet0, b_bet0),
            (W_lin1, b_lin1, W_gam1, b_gam1, W_bet1, b_bet1),
            (W_lin2, b_lin2, W_gam2, b_gam2, W_bet2, b_bet2)):
        Wcat = jnp.concatenate([Wl.T, Wg.T, Wb.T], axis=1)
        bcat = jnp.concatenate([bl, bg, bb]).reshape(1, 3 * D)
        params.append((Wcat, bcat))

    xl2, gam, bet = _proj_call(x, params[0][0], params[0][1])
    agg2 = _edge_call(xl2, src_r, dst_r, zeros)
    for i in (1, 2):
        xl2, gam, bet = _film_proj_call(gam, bet, agg2,
                                        params[i][0], params[i][1])
        agg2 = _edge_call(xl2, src_r, dst_r, zeros)
    return _film_pool_call(gam, bet, agg2, batch3)


# speedup vs baseline: 1.2506x; 1.2506x over previous
import functools

import jax
import jax.numpy as jnp
from jax import lax
from jax.experimental import pallas as pl
from jax.experimental.pallas import tpu as pltpu
from jax.experimental.pallas import tpu_sc as plsc

N = 10000
E = 160000
D = 256
G = 32
DH = 128
NC = 2
NS = 16
CH = 128
ROWS = E // CH
NBUF = 2
RQ = -(-(-(-ROWS // NS)) // NBUF) * NBUF
ROWS_PAD = RQ * NS
NPAD = 16
ZB = (N // NS) // 8 * 8
ZREM = N - ZB * NS
BN = 1000
GRID = N // BN


def _proj_body(h_ref, w_ref, b_ref, xl2_ref, gam_ref, bet_ref):
    h = h_ref[...]
    o = jnp.dot(h, w_ref[...], preferred_element_type=jnp.float32) + b_ref[...]
    xl2_ref[0] = o[:, :DH]
    xl2_ref[1] = o[:, DH:D]
    gam_ref[...] = o[:, D:D + D]
    bet_ref[...] = o[:, D + D:]


def _film_proj_body(gamp_ref, betp_ref, agg_ref, w_ref, b_ref,
                    xl2_ref, gam_ref, bet_ref):
    agg = jnp.concatenate([agg_ref[0], agg_ref[1]], axis=-1)
    h = jnp.maximum(gamp_ref[...] * agg + betp_ref[...], 0.0)
    o = jnp.dot(h, w_ref[...], preferred_element_type=jnp.float32) + b_ref[...]
    xl2_ref[0] = o[:, :DH]
    xl2_ref[1] = o[:, DH:D]
    gam_ref[...] = o[:, D:D + D]
    bet_ref[...] = o[:, D + D:]


def _film_pool_body(gamp_ref, betp_ref, agg_ref, batch_ref, out_ref,
                    sums_scr, counts_scr):
    i = pl.program_id(0)
    agg = jnp.concatenate([agg_ref[0], agg_ref[1]], axis=-1)
    h = jnp.maximum(gamp_ref[...] * agg + betp_ref[...], 0.0)
    bvec = batch_ref[0]
    gids = lax.broadcasted_iota(jnp.int32, (G, BN), 0)
    onehot = (gids == bvec).astype(jnp.float32)
    ps = jnp.dot(onehot, h, preferred_element_type=jnp.float32)
    pc = jnp.dot(onehot, jnp.ones((BN, D), jnp.float32),
                 preferred_element_type=jnp.float32)

    @pl.when(i == 0)
    def _():
        sums_scr[...] = ps
        counts_scr[...] = pc

    @pl.when(i > 0)
    def _():
        sums_scr[...] += ps
        counts_scr[...] += pc

    @pl.when(i == pl.num_programs(0) - 1)
    def _():
        out_ref[...] = sums_scr[...] / jnp.maximum(counts_scr[...], 1.0)


_W_SPEC = pl.BlockSpec((D, 3 * D), lambda i: (0, 0))
_B_SPEC = pl.BlockSpec((1, 3 * D), lambda i: (0, 0))
_H_SPEC = pl.BlockSpec((BN, D), lambda i: (i, 0))
_XL2_SPEC = pl.BlockSpec((2, BN, DH), lambda i: (0, i, 0))

_proj_call = pl.pallas_call(
    _proj_body,
    grid=(GRID,),
    in_specs=[_H_SPEC, _W_SPEC, _B_SPEC],
    out_specs=[_XL2_SPEC, _H_SPEC, _H_SPEC],
    out_shape=[
        jax.ShapeDtypeStruct((2, N, DH), jnp.float32),
        jax.ShapeDtypeStruct((N, D), jnp.float32),
        jax.ShapeDtypeStruct((N, D), jnp.float32),
    ],
)

_film_proj_call = pl.pallas_call(
    _film_proj_body,
    grid=(GRID,),
    in_specs=[_H_SPEC, _H_SPEC, _XL2_SPEC, _W_SPEC, _B_SPEC],
    out_specs=[_XL2_SPEC, _H_SPEC, _H_SPEC],
    out_shape=[
        jax.ShapeDtypeStruct((2, N, DH), jnp.float32),
        jax.ShapeDtypeStruct((N, D), jnp.float32),
        jax.ShapeDtypeStruct((N, D), jnp.float32),
    ],
)

_film_pool_call = pl.pallas_call(
    _film_pool_body,
    grid=(GRID,),
    in_specs=[_H_SPEC, _H_SPEC, _XL2_SPEC,
              pl.BlockSpec((1, 1, BN), lambda i: (i, 0, 0))],
    out_specs=pl.BlockSpec((G, D), lambda i: (0, 0)),
    out_shape=jax.ShapeDtypeStruct((G, D), jnp.float32),
    scratch_shapes=[pltpu.VMEM((G, D), jnp.float32),
                    pltpu.VMEM((G, D), jnp.float32)],
)


def _edge_body(xl_hbm, srcr_hbm, dstr_hbm, zeros_hbm, out_hbm,
               sidx, didx_all, rows, acc, isem, gsem, zsem):
    c = lax.axis_index("c")
    s = lax.axis_index("s")

    pltpu.async_copy(zeros_hbm.at[pl.ds(0, ZB)], acc.at[pl.ds(s * ZB, ZB)],
                     zsem)

    @pl.when(s == NS - 1)
    def _():
        pltpu.async_copy(zeros_hbm.at[pl.ds(0, ZREM + NPAD)],
                         acc.at[pl.ds(N - ZREM, ZREM + NPAD)], zsem)

    rowbase = pl.multiple_of(s * RQ, 8)
    pltpu.sync_copy(dstr_hbm.at[pl.ds(rowbase, RQ)], didx_all)

    for b in range(NBUF):
        pltpu.async_copy(srcr_hbm.at[pl.ds(rowbase + b, 1)],
                         sidx[b], isem[b])
    pltpu.make_async_copy(srcr_hbm.at[pl.ds(rowbase, 1)],
                          sidx[0], isem[0]).wait()
    pltpu.async_copy(xl_hbm.at[c].at[sidx[0].at[0]], rows.at[0], gsem[0])

    pltpu.make_async_copy(zeros_hbm.at[pl.ds(0, ZB)],
                          acc.at[pl.ds(s * ZB, ZB)], zsem).wait()

    @pl.when(s == NS - 1)
    def _():
        pltpu.make_async_copy(zeros_hbm.at[pl.ds(0, ZREM + NPAD)],
                              acc.at[pl.ds(N - ZREM, ZREM + NPAD)],
                              zsem).wait()

    plsc.subcore_barrier()

    def body(g, carry):
        for b in range(NBUF):
            j = g * NBUF + b
            bn = (b + 1) % NBUF
            jn = j + 1

            @pl.when(jn < RQ)
            def _():
                pltpu.make_async_copy(srcr_hbm.at[pl.ds(rowbase + jn, 1)],
                                      sidx[bn], isem[bn]).wait()
                pltpu.async_copy(xl_hbm.at[c].at[sidx[bn].at[0]],
                                 rows.at[bn], gsem[bn])

            pltpu.make_async_copy(xl_hbm.at[c].at[sidx[b].at[0]],
                                  rows.at[b], gsem[b]).wait()
            pltpu.sync_copy(rows.at[b], acc.at[didx_all.at[j]], add=True)
            jn2 = j + 2

            @pl.when(jn2 < RQ)
            def _():
                pltpu.async_copy(srcr_hbm.at[pl.ds(rowbase + jn2, 1)],
                                 sidx[b], isem[b])
        return carry

    lax.fori_loop(0, RQ // NBUF, body, 0)

    plsc.subcore_barrier()

    pltpu.sync_copy(acc.at[pl.ds(s * ZB, ZB)],
                    out_hbm.at[c, pl.ds(s * ZB, ZB)])

    @pl.when(s == NS - 1)
    def _():
        pltpu.sync_copy(acc.at[pl.ds(N - ZREM, ZREM)],
                        out_hbm.at[c, pl.ds(N - ZREM, ZREM)])


@functools.cache
def _get_edge_call():
    return pl.kernel(
        _edge_body,
        out_type=jax.ShapeDtypeStruct((2, N, DH), jnp.float32),
        mesh=plsc.VectorSubcoreMesh(core_axis_name="c", subcore_axis_name="s",
                                    num_cores=NC, num_subcores=NS),
        scratch_types=[
            [pltpu.VMEM((1, CH), jnp.int32)] * NBUF,
            pltpu.VMEM((RQ, CH), jnp.int32),
            pltpu.VMEM((NBUF, CH, DH), jnp.float32),
            pltpu.VMEM_SHARED((N + NPAD, DH), jnp.float32),
            [pltpu.SemaphoreType.DMA] * NBUF,
            [pltpu.SemaphoreType.DMA] * NBUF,
            pltpu.SemaphoreType.DMA,
        ],
    )


def _edge_call(xl2, src_r, dst_r, zeros):
    return _get_edge_call()(xl2, src_r, dst_r, zeros)


def kernel(x, edge_index, batch,
           W_lin0, b_lin0, W_gam0, b_gam0, W_bet0, b_bet0,
           W_lin1, b_lin1, W_gam1, b_gam1, W_bet1, b_bet1,
           W_lin2, b_lin2, W_gam2, b_gam2, W_bet2, b_bet2):
    npad_e = ROWS_PAD * CH - E
    src = edge_index[0].astype(jnp.int32)
    dst = edge_index[1].astype(jnp.int32)
    pad_src = jnp.zeros((npad_e,), jnp.int32)
    pad_dst = N + jnp.arange(npad_e, dtype=jnp.int32) % NPAD
    src_r = jnp.concatenate([src, pad_src]).reshape(ROWS_PAD, CH)
    dst_r = jnp.concatenate([dst, pad_dst]).reshape(ROWS_PAD, CH)
    zeros = jnp.zeros((ZB, DH), jnp.float32)
    batch3 = batch.astype(jnp.int32).reshape(GRID, 1, BN)

    params = []
    for (Wl, bl, Wg, bg, Wb, bb) in (
            (W_lin0, b_lin0, W_gam0, b_gam0, W_bet0, b_bet0),
            (W_lin1, b_lin1, W_gam1, b_gam1, W_bet1, b_bet1),
            (W_lin2, b_lin2, W_gam2, b_gam2, W_bet2, b_bet2)):
        Wcat = jnp.concatenate([Wl.T, Wg.T, Wb.T], axis=1)
        bcat = jnp.concatenate([bl, bg, bb]).reshape(1, 3 * D)
        params.append((Wcat, bcat))

    xl2, gam, bet = _proj_call(x, params[0][0], params[0][1])
    agg2 = _edge_call(xl2, src_r, dst_r, zeros)
    for i in (1, 2):
        xl2, gam, bet = _film_proj_call(gam, bet, agg2,
                                        params[i][0], params[i][1])
        agg2 = _edge_call(xl2, src_r, dst_r, zeros)
    return _film_pool_call(gam, bet, agg2, batch3)
